# Initial kernel scaffold; baseline (speedup 1.0000x reference)
import jax
import jax.numpy as jnp
from jax.experimental import pallas as pl


def kernel(inputs, u_indices, v_indices, weights, weights_scalars, user_bias, item_bias):
    raise NotImplementedError("write your pallas kernel here")


# speedup vs baseline: 3.2005x; 3.2005x over previous
import functools

import jax
import jax.numpy as jnp
from jax import lax
from jax.experimental import pallas as pl
from jax.experimental.pallas import tpu as pltpu
from jax.experimental.pallas import tpu_sc as plsc

_NUSERS = 1_000_000
_D = 32
_NCLS = 5
_NBASIS = 3
_E = 2_000_000
_K = 128
_NCP = 8


def _make_gather():
    info = plsc.get_sparse_core_info()
    ncores, nsub = info.num_cores, info.num_subcores
    nw = ncores * nsub
    nchunks_half = _E // _K
    rounds, rem = divmod(nchunks_half, nw)
    mesh = plsc.VectorSubcoreMesh(core_axis_name="c", subcore_axis_name="s")

    @functools.partial(
        pl.kernel,
        mesh=mesh,
        compiler_params=pltpu.CompilerParams(use_tc_tiling_on_sc=False),
        out_type=[
            jax.ShapeDtypeStruct((2 * _E, _D), jnp.float32),
            jax.ShapeDtypeStruct((2 * _E, _NCP), jnp.float32),
        ],
        scratch_types=[
            pltpu.VMEM((_K,), jnp.int32),
            pltpu.VMEM((_K,), jnp.int32),
            pltpu.VMEM((_K, _D), jnp.float32),
            pltpu.VMEM((_K, _NCP), jnp.float32),
            pltpu.SemaphoreType.DMA,
            pltpu.SemaphoreType.DMA,
        ],
    )
    def gather_kernel(table, ubias, ibias, u_idx, v_idx,
                      rows_out, brows_out,
                      idx_v, idx_s, rows_v, brows_v, sem_r, sem_b):
        wid = lax.axis_index("s") * ncores + lax.axis_index("c")

        def do_chunk(g, idx_ref, bias_ref, out_base, shift):
            base = g * _K
            pltpu.sync_copy(idx_ref.at[pl.ds(base, _K)], idx_v)
            if shift:
                for i in range(_K // 16):
                    sl = pl.ds(i * 16, 16)
                    idx_s[sl] = idx_v[sl] + _NUSERS
                tbl_idx = idx_s
            else:
                tbl_idx = idx_v
            cp_r = pltpu.async_copy(table.at[tbl_idx], rows_v, sem_r)
            cp_b = pltpu.async_copy(bias_ref.at[idx_v], brows_v, sem_b)
            cp_r.wait()
            cp_b.wait()
            pltpu.sync_copy(rows_v, rows_out.at[pl.ds(out_base + base, _K)])
            pltpu.sync_copy(brows_v, brows_out.at[pl.ds(out_base + base, _K)])

        def half(idx_ref, bias_ref, out_base, shift):
            def body(r, _):
                do_chunk(r * nw + wid, idx_ref, bias_ref, out_base, shift)
                return 0

            lax.fori_loop(0, rounds, body, 0)

            @pl.when(wid < rem)
            def _():
                do_chunk(rounds * nw + wid, idx_ref, bias_ref, out_base, shift)

        half(u_idx, ubias, 0, False)
        half(v_idx, ibias, _E, True)

    return gather_kernel


_gather = _make_gather()


def _make_compute(blk):
    grid = _E // blk
    voff = _E // blk

    def tc_kernel(u_ref, v_ref, ub_ref, vb_ref, wcat_ref, wsel_ref, out_ref):
        v = v_ref[...]
        uw = jnp.dot(u_ref[...], wcat_ref[...],
                     preferred_element_type=jnp.float32)
        p = uw * jnp.concatenate([v, v, v], axis=1)
        z = jnp.dot(p, wsel_ref[...], preferred_element_type=jnp.float32)
        z = z + ub_ref[:, :_NCLS] + vb_ref[:, :_NCLS]
        z = z - jnp.max(z, axis=1, keepdims=True)
        ez = jnp.exp(z)
        out_ref[...] = ez / jnp.sum(ez, axis=1, keepdims=True)

    return pl.pallas_call(
        tc_kernel,
        grid=(grid,),
        in_specs=[
            pl.BlockSpec((blk, _D), lambda b: (b, 0)),
            pl.BlockSpec((blk, _D), lambda b: (voff + b, 0)),
            pl.BlockSpec((blk, _NCP), lambda b: (b, 0)),
            pl.BlockSpec((blk, _NCP), lambda b: (voff + b, 0)),
            pl.BlockSpec((_D, _D * _NBASIS), lambda b: (0, 0)),
            pl.BlockSpec((_D * _NBASIS, _NCLS), lambda b: (0, 0)),
        ],
        out_specs=pl.BlockSpec((blk, _NCLS), lambda b: (b, 0)),
        out_shape=jax.ShapeDtypeStruct((_E, _NCLS), jnp.float32),
    )


_compute = _make_compute(4000)


def kernel(inputs, u_indices, v_indices, weights, weights_scalars,
           user_bias, item_bias):
    table = inputs.reshape(2 * _NUSERS, _D)
    u_idx = u_indices.astype(jnp.int32)
    v_idx = v_indices.astype(jnp.int32)
    wcat = jnp.concatenate([weights[i] for i in range(_NBASIS)], axis=1)
    wsel = jnp.repeat(weights_scalars, _D, axis=0)
    ubias_p = jnp.pad(user_bias, ((0, 0), (0, _NCP - _NCLS)))
    ibias_p = jnp.pad(item_bias, ((0, 0), (0, _NCP - _NCLS)))
    rows, brows = _gather(table, ubias_p, ibias_p, u_idx, v_idx)
    return _compute(rows, rows, brows, brows, wcat, wsel)
